# baseline (speedup 1.0000x reference)
import functools

import jax
import jax.numpy as jnp
from jax import lax
from jax.experimental import pallas as pl
from jax.experimental.pallas import tpu as pltpu
from jax.experimental.pallas import tpu_sc as plsc

N = 6890
NW = 32
B = 224
CH = 112
PAD = NW * B
NVEC = B // 16
NCH = B // CH
NBLK = (N + 127) // 128
LASTC = (NBLK - 1) * 128
RING = 3

_mesh = plsc.VectorSubcoreMesh(core_axis_name="c", subcore_axis_name="s")


@functools.partial(
    pl.kernel,
    mesh=_mesh,
    out_type=jax.ShapeDtypeStruct((NW, 16), jnp.float32),
    compiler_params=pltpu.CompilerParams(needs_layout_passes=False),
    scratch_types=[
        pltpu.VMEM((B,), jnp.int32),
        pltpu.VMEM((B,), jnp.int32),
        pltpu.VMEM((B,), jnp.int32),
        pltpu.VMEM((B,), jnp.int32),
        pltpu.VMEM((B,), jnp.int32),
        pltpu.VMEM((B,), jnp.int32),
        pltpu.VMEM((B,), jnp.int32),
        pltpu.VMEM((B, 128), jnp.float32),
        pltpu.VMEM((B, 128), jnp.float32),
        pltpu.VMEM((B, 128), jnp.float32),
        pltpu.VMEM((16,), jnp.float32),
        pltpu.SemaphoreType.DMA,
        pltpu.SemaphoreType.DMA,
        pltpu.SemaphoreType.DMA,
        pltpu.SemaphoreType.DMA,
    ],
)
def _gather_partials(p2p_hbm, dist_hbm, last_hbm, sc_hbm, tc_hbm, out_hbm,
                     idx_v, row_v, blk_v, lane_v, bidx0, bidx1, bidx2,
                     strips0, strips1, strips2, acc_v,
                     sem0, sem1, sem2, sem_s):
    sems = (sem0, sem1, sem2)
    bidxs = (bidx0, bidx1, bidx2)
    strips = (strips0, strips1, strips2)
    wid = lax.axis_index("s") * 2 + lax.axis_index("c")
    base = wid * B

    pltpu.sync_copy(tc_hbm.at[pl.ds(base, B)], idx_v)
    copies = [
        pltpu.async_copy(p2p_hbm.at[idx_v.at[pl.ds(h * CH, CH)]],
                         row_v.at[pl.ds(h * CH, CH)], sem_s)
        for h in range(NCH)
    ]
    pltpu.sync_copy(sc_hbm.at[pl.ds(base, B)], lane_v)
    for c in copies:
        c.wait()

    lane16 = lax.iota(jnp.int32, 16)
    for i in range(NVEC):
        sl = pl.ds(i * 16, 16)
        s = lane_v[sl]
        blk_v[sl] = s >> 7
        lane_v[sl] = s & 127

    def fire_block(q, k, last=False):
        for i in range(NVEC):
            sl = pl.ds(i * 16, 16)
            bidxs[k][sl] = jnp.where(blk_v[sl] == q, row_v[sl], 0)
        cps = []
        for h in range(NCH):
            idx = bidxs[k].at[pl.ds(pl.multiple_of(h * CH, 8), CH)]
            if last:
                src = last_hbm.at[idx]
            else:
                src = dist_hbm.at[idx,
                                  pl.ds(pl.multiple_of(q * 128, 128), 128)]
            cps.append(pltpu.async_copy(src, strips[k].at[pl.ds(h * CH, CH)],
                                        sems[k]))
        return cps

    def extract_block(q, k, acc):
        for i in range(NVEC):
            sl = pl.ds(i * 16, 16)
            vals = plsc.load_gather(strips[k], [lane16 + i * 16, lane_v[sl]])
            g = lane16 + (base + i * 16)
            m = (blk_v[sl] == q) & (g < N)
            acc = acc + jnp.where(m, vals, 0.0)
        return acc

    def run_triad(qs, acc, last=False):
        allcps = [fire_block(q, k, last=(last and k == RING - 1))
                  for k, q in enumerate(qs)]
        for k, q in enumerate(qs):
            for c in allcps[k]:
                c.wait()
            acc = extract_block(q, k, acc)
        return acc

    acc0 = jnp.zeros((16,), jnp.float32)

    @pl.loop(0, (NBLK - RING) // RING, init_carry=acc0)
    def _triad(t, acc):
        return run_triad([t * RING + k for k in range(RING)], acc)

    acc = run_triad([NBLK - 3, NBLK - 2, NBLK - 1], _triad, last=True)

    acc_v[...] = acc
    pltpu.sync_copy(acc_v, out_hbm.at[wid])


def _mean_body(x_ref, o_ref):
    o_ref[...] = jnp.sum(x_ref[...], keepdims=True).reshape(1, 1) * (1.0 / N)


_mean_call = pl.pallas_call(
    _mean_body,
    out_shape=jax.ShapeDtypeStruct((1, 1), jnp.float32),
)


def kernel(p2p21, source_distances, source_corr, target_corr):
    p2p = p2p21.astype(jnp.int32)
    tc = jnp.pad(target_corr.astype(jnp.int32), (0, PAD - N))
    sc = jnp.pad(source_corr.astype(jnp.int32), (0, PAD - N))
    last = jnp.pad(source_distances[:, LASTC:], ((0, 0), (0, NBLK * 128 - N)))
    partials = _gather_partials(p2p, source_distances, last, sc, tc)
    return _mean_call(partials)[0, 0]


# speedup vs baseline: 5.8086x; 5.8086x over previous
import functools

import jax
import jax.numpy as jnp
from jax import lax
from jax.experimental import pallas as pl
from jax.experimental.pallas import tpu as pltpu
from jax.experimental.pallas import tpu_sc as plsc

N = 6890
NW = 32
B = 224
CH = 112
PAD = NW * B
NVEC = B // 16
NCH = B // CH

_mesh = plsc.VectorSubcoreMesh(core_axis_name="c", subcore_axis_name="s")


@functools.partial(
    pl.kernel,
    mesh=_mesh,
    out_type=jax.ShapeDtypeStruct((NW, 16), jnp.float32),
    scratch_types=[
        pltpu.VMEM((B,), jnp.int32),
        pltpu.VMEM((B,), jnp.int32),
        pltpu.VMEM((B,), jnp.int32),
        pltpu.VMEM((B,), jnp.int32),
        pltpu.VMEM((B,), jnp.float32),
        pltpu.VMEM((16,), jnp.float32),
        pltpu.SemaphoreType.DMA,
    ],
)
def _gather_partials(p2p_hbm, dist_hbm, sc_hbm, tc_hbm, out_hbm,
                     idx_v, sc_v, map_v, flat_v, vals_v, acc_v, sem):
    wid = lax.axis_index("s") * 2 + lax.axis_index("c")
    base = wid * B

    pltpu.sync_copy(tc_hbm.at[pl.ds(base, B)], idx_v)
    copies = [
        pltpu.async_copy(p2p_hbm.at[idx_v.at[pl.ds(h * CH, CH)]],
                         map_v.at[pl.ds(h * CH, CH)], sem)
        for h in range(NCH)
    ]
    pltpu.sync_copy(sc_hbm.at[pl.ds(base, B)], sc_v)
    for c in copies:
        c.wait()

    for i in range(NVEC):
        sl = pl.ds(i * 16, 16)
        flat_v[sl] = map_v[sl] * N + sc_v[sl]

    copies = [
        pltpu.async_copy(dist_hbm.at[flat_v.at[pl.ds(h * CH, CH)]],
                         vals_v.at[pl.ds(h * CH, CH)], sem)
        for h in range(NCH)
    ]
    for c in copies:
        c.wait()

    lane = lax.iota(jnp.int32, 16)
    acc = jnp.zeros((16,), jnp.float32)
    for i in range(NVEC):
        g = lane + (base + i * 16)
        acc = acc + jnp.where(g < N, vals_v[pl.ds(i * 16, 16)], 0.0)

    acc_v[...] = acc
    pltpu.sync_copy(acc_v, out_hbm.at[wid])


def _mean_body(x_ref, o_ref):
    o_ref[...] = jnp.sum(x_ref[...], keepdims=True).reshape(1, 1) * (1.0 / N)


_mean_call = pl.pallas_call(
    _mean_body,
    out_shape=jax.ShapeDtypeStruct((1, 1), jnp.float32),
)


def kernel(p2p21, source_distances, source_corr, target_corr):
    p2p = p2p21.astype(jnp.int32)
    tc = jnp.pad(target_corr.astype(jnp.int32), (0, PAD - N))
    sc = jnp.pad(source_corr.astype(jnp.int32), (0, PAD - N))
    dist = source_distances.reshape(-1)
    partials = _gather_partials(p2p, dist, sc, tc)
    return _mean_call(partials)[0, 0]
